# baseline (speedup 1.0000x reference)
import functools

import jax
import jax.numpy as jnp
from jax import lax
from jax.experimental import pallas as pl
from jax.experimental.pallas import tpu as pltpu
from jax.experimental.pallas import tpu_sc as plsc

D_MODEL = 768
SEQ = 512
VOCAB = 2 * SEQ + 1
N_CORES = 2
N_SUBCORES = 16
N_WORKERS = N_CORES * N_SUBCORES
I_PER_W = SEQ // N_WORKERS
N_PAIRS = I_PER_W // 2

CH = 64
N_CHUNKS = SEQ // CH
GROWS = CH + 8
TROWS = 1032


def _sc_copy(table8_hbm, out_hbm, buf0, buf1, gsem, ssem0, ssem1):
    wid = lax.axis_index("s") * N_CORES + lax.axis_index("c")
    base_i = wid * I_PER_W
    win0 = pl.multiple_of(SEQ - base_i, 16)
    bufs = (buf0, buf1)
    ssems = (ssem0, ssem1)

    def gather(u):
        c, t = divmod(u, N_PAIRS)
        start = TROWS * t + win0 + c * CH - t - 8 + t
        return pltpu.async_copy(
            table8_hbm.at[pl.ds(pl.multiple_of(start, 8), GROWS)],
            bufs[u % 2],
            gsem,
        )

    NU = N_CHUNKS * N_PAIRS
    pending = {}
    g = gather(0)
    for u in range(NU):
        c, t = divmod(u, N_PAIRS)
        b = u % 2
        g.wait()
        handles = [
            pltpu.async_copy(
                bufs[b].at[pl.ds(8, CH)],
                out_hbm.at[base_i + t, pl.ds(c * CH, CH), :],
                ssems[b],
            ),
            pltpu.async_copy(
                bufs[b].at[pl.ds(0, CH)],
                out_hbm.at[base_i + t + 8, pl.ds(c * CH, CH), :],
                ssems[b],
            ),
        ]
        pending[u] = handles
        if u + 1 < NU:
            if u - 1 >= 0:
                for h in pending.pop(u - 1):
                    h.wait()
            g = gather(u + 1)
    for hs in pending.values():
        for h in hs:
            h.wait()


def kernel(seq_len, emb):
    del seq_len
    emb_rev = emb[::-1]
    pad8 = jnp.zeros((8, D_MODEL), jnp.float32)
    table8 = jnp.concatenate([emb_rev, pad8] * 8, axis=0)
    mesh = plsc.VectorSubcoreMesh(core_axis_name="c", subcore_axis_name="s")
    return pl.kernel(
        _sc_copy,
        mesh=mesh,
        out_type=jax.ShapeDtypeStruct((SEQ, SEQ, D_MODEL), jnp.float32),
        scratch_types=[
            pltpu.VMEM((GROWS, D_MODEL), jnp.float32),
            pltpu.VMEM((GROWS, D_MODEL), jnp.float32),
            pltpu.SemaphoreType.DMA,
            pltpu.SemaphoreType.DMA,
            pltpu.SemaphoreType.DMA,
        ],
    )(table8)


# speedup vs baseline: 1.0408x; 1.0408x over previous
import functools

import jax
import jax.numpy as jnp
from jax import lax
from jax.experimental import pallas as pl
from jax.experimental.pallas import tpu as pltpu
from jax.experimental.pallas import tpu_sc as plsc

D_MODEL = 768
SEQ = 512
VOCAB = 2 * SEQ + 1
N_CORES = 2
N_SUBCORES = 16
N_WORKERS = N_CORES * N_SUBCORES
I_PER_W = SEQ // N_WORKERS
N_PAIRS = I_PER_W // 2

CH = 64
N_CHUNKS = SEQ // CH
GROWS = CH + 8
TROWS = 1032


def _sc_copy(table8_hbm, out_hbm, buf0, buf1, gsem, ssem0, ssem1):
    wid = lax.axis_index("s") * N_CORES + lax.axis_index("c")
    base_i = wid * I_PER_W
    win0 = pl.multiple_of(SEQ - base_i, 16)
    bufs = (buf0, buf1)
    ssems = (ssem0, ssem1)

    def gather(u):
        c, t = divmod(u, N_PAIRS)
        start = TROWS * t + win0 + c * CH - t - 8 + t
        return pltpu.async_copy(
            table8_hbm.at[pl.ds(pl.multiple_of(start, 8), GROWS)],
            bufs[u % 2],
            gsem,
        )

    NU = N_CHUNKS * N_PAIRS
    pending = {}
    g = gather(0)
    for u in range(NU):
        c, t = divmod(u, N_PAIRS)
        b = u % 2
        g.wait()
        handles = [
            pltpu.async_copy(
                bufs[b].at[pl.ds(8, CH)],
                out_hbm.at[base_i + t, pl.ds(c * CH, CH), :],
                ssems[b],
            ),
            pltpu.async_copy(
                bufs[b].at[pl.ds(0, CH)],
                out_hbm.at[base_i + t + 8, pl.ds(c * CH, CH), :],
                ssems[b],
            ),
        ]
        pending[u] = handles
        if u + 1 < NU:
            if u - 1 >= 0:
                for h in pending.pop(u - 1):
                    h.wait()
            g = gather(u + 1)
    for hs in pending.values():
        for h in hs:
            h.wait()


def kernel(seq_len, emb):
    del seq_len
    emb_rev = emb[::-1]
    table8 = jnp.concatenate(
        [jnp.broadcast_to(emb_rev[None], (8, VOCAB, D_MODEL)),
         jnp.zeros((8, 8, D_MODEL), jnp.float32)], axis=1,
    ).reshape(8 * (VOCAB + 8), D_MODEL)
    mesh = plsc.VectorSubcoreMesh(core_axis_name="c", subcore_axis_name="s")
    return pl.kernel(
        _sc_copy,
        mesh=mesh,
        out_type=jax.ShapeDtypeStruct((SEQ, SEQ, D_MODEL), jnp.float32),
        scratch_types=[
            pltpu.VMEM((GROWS, D_MODEL), jnp.float32),
            pltpu.VMEM((GROWS, D_MODEL), jnp.float32),
            pltpu.SemaphoreType.DMA,
            pltpu.SemaphoreType.DMA,
            pltpu.SemaphoreType.DMA,
        ],
    )(table8)


# speedup vs baseline: 1.1577x; 1.1123x over previous
import functools

import jax
import jax.numpy as jnp
from jax import lax
from jax.experimental import pallas as pl
from jax.experimental.pallas import tpu as pltpu
from jax.experimental.pallas import tpu_sc as plsc

D_MODEL = 768
SEQ = 512
VOCAB = 2 * SEQ + 1
N_CORES = 2
N_SUBCORES = 16
N_WORKERS = N_CORES * N_SUBCORES
I_PER_W = SEQ // N_WORKERS
N_PAIRS = I_PER_W // 2

GSIZE = 4
HALO = 8 * (GSIZE - 1)
TROWS = VOCAB + 8
CHUNKS = tuple([(c * 48, 48) for c in range(10)] + [(480, 32)])
GMAX = 48 + HALO


def _sc_copy(table8_hbm, out_hbm, buf0, buf1, gsem, ssem0, ssem1):
    wid = lax.axis_index("s") * N_CORES + lax.axis_index("c")
    rng = wid // 2
    half = wid % 2
    bufs = (buf0, buf1)
    ssems = (ssem0, ssem1)
    units = [(g, ci) for g in range(GSIZE) for ci in range(len(CHUNKS))]

    def gather(unit):
        g, ci = unit
        c0, _ch = CHUNKS[ci]
        m = GSIZE * half + g
        start = TROWS * m + (SEQ - HALO + c0) - 32 * rng - m
        return pltpu.async_copy(
            table8_hbm.at[pl.ds(pl.multiple_of(start, 8), _ch + HALO)],
            bufs[units.index(unit) % 2].at[pl.ds(0, _ch + HALO)],
            gsem,
        )

    pending = {}
    g = gather(units[0])
    for ui, unit in enumerate(units):
        gidx, ci = unit
        c0, ch = CHUNKS[ci]
        b = ui % 2
        g.wait()
        m = GSIZE * half + gidx
        handles = []
        for j in range(GSIZE):
            handles.append(pltpu.async_copy(
                bufs[b].at[pl.ds(HALO - 8 * j, ch)],
                out_hbm.at[32 * rng + m + 8 * j, pl.ds(c0, ch), :],
                ssems[b],
            ))
        pending[ui] = handles
        if ui + 1 < len(units):
            if ui - 1 >= 0:
                for h in pending.pop(ui - 1):
                    h.wait()
            g = gather(units[ui + 1])
    for hs in pending.values():
        for h in hs:
            h.wait()


def kernel(seq_len, emb):
    del seq_len
    emb_rev = emb[::-1]
    table8 = jnp.concatenate(
        [jnp.broadcast_to(emb_rev[None], (8, VOCAB, D_MODEL)),
         jnp.zeros((8, 8, D_MODEL), jnp.float32)], axis=1,
    ).reshape(8 * (VOCAB + 8), D_MODEL)
    mesh = plsc.VectorSubcoreMesh(core_axis_name="c", subcore_axis_name="s")
    return pl.kernel(
        _sc_copy,
        mesh=mesh,
        out_type=jax.ShapeDtypeStruct((SEQ, SEQ, D_MODEL), jnp.float32),
        scratch_types=[
            pltpu.VMEM((GMAX, D_MODEL), jnp.float32),
            pltpu.VMEM((GMAX, D_MODEL), jnp.float32),
            pltpu.SemaphoreType.DMA,
            pltpu.SemaphoreType.DMA,
            pltpu.SemaphoreType.DMA,
        ],
    )(table8)


# speedup vs baseline: 1.1815x; 1.0206x over previous
import functools

import jax
import jax.numpy as jnp
from jax import lax
from jax.experimental import pallas as pl
from jax.experimental.pallas import tpu as pltpu
from jax.experimental.pallas import tpu_sc as plsc

D_MODEL = 768
SEQ = 512
VOCAB = 2 * SEQ + 1
N_CORES = 2
N_SUBCORES = 16
N_WORKERS = N_CORES * N_SUBCORES
I_PER_W = SEQ // N_WORKERS
N_PAIRS = I_PER_W // 2

GSIZE = 4
HALO = 8 * (GSIZE - 1)
TROWS = VOCAB + 8
CHUNKS = tuple([(c * 48, 48) for c in range(10)] + [(480, 32)])
GMAX = 48 + HALO


def _sc_copy(table8_hbm, out_hbm, buf0, buf1, gsem, ssem0, ssem1):
    wid = lax.axis_index("s") * N_CORES + lax.axis_index("c")
    rng = wid // 2
    half = wid % 2
    bufs = (buf0, buf1)
    ssems = (ssem0, ssem1)
    units = [(g, ci) for g in range(GSIZE) for ci in range(len(CHUNKS))]

    def gather(unit):
        g, ci = unit
        c0, _ch = CHUNKS[ci]
        m = GSIZE * half + g
        start = TROWS * m + (SEQ - HALO + c0) - 32 * rng - m
        return pltpu.async_copy(
            table8_hbm.at[pl.ds(pl.multiple_of(start, 8), _ch + HALO)],
            bufs[units.index(unit) % 2].at[pl.ds(0, _ch + HALO)],
            gsem,
        )

    pending = {}
    g = gather(units[0])
    for ui, unit in enumerate(units):
        gidx, ci = unit
        c0, ch = CHUNKS[ci]
        b = ui % 2
        g.wait()
        m = GSIZE * half + gidx
        handles = []
        for j in range(GSIZE):
            handles.append(pltpu.async_copy(
                bufs[b].at[pl.ds(HALO - 8 * j, ch)],
                out_hbm.at[32 * rng + m + 8 * j, pl.ds(c0, ch), :],
                ssems[b],
            ))
        pending[ui] = handles
        if ui + 1 < len(units):
            if ui - 1 >= 0:
                for h in pending.pop(ui - 1):
                    h.wait()
            g = gather(units[ui + 1])
    for hs in pending.values():
        for h in hs:
            h.wait()


def kernel(seq_len, emb):
    del seq_len
    emb_rev = emb[::-1]
    unit = jnp.concatenate([emb_rev, emb_rev[:8]], axis=0)
    table8 = jnp.broadcast_to(unit[None], (8, TROWS, D_MODEL)).reshape(
        8 * TROWS, D_MODEL)
    mesh = plsc.VectorSubcoreMesh(core_axis_name="c", subcore_axis_name="s")
    return pl.kernel(
        _sc_copy,
        mesh=mesh,
        out_type=jax.ShapeDtypeStruct((SEQ, SEQ, D_MODEL), jnp.float32),
        scratch_types=[
            pltpu.VMEM((GMAX, D_MODEL), jnp.float32),
            pltpu.VMEM((GMAX, D_MODEL), jnp.float32),
            pltpu.SemaphoreType.DMA,
            pltpu.SemaphoreType.DMA,
            pltpu.SemaphoreType.DMA,
        ],
    )(table8)


# speedup vs baseline: 1.2317x; 1.0425x over previous
import functools

import jax
import jax.numpy as jnp
from jax import lax
from jax.experimental import pallas as pl
from jax.experimental.pallas import tpu as pltpu
from jax.experimental.pallas import tpu_sc as plsc

D_MODEL = 768
SEQ = 512
VOCAB = 2 * SEQ + 1
N_CORES = 2
N_SUBCORES = 16
N_WORKERS = N_CORES * N_SUBCORES
I_PER_W = SEQ // N_WORKERS
N_PAIRS = I_PER_W // 2

GSIZE = 4
HALO = 8 * (GSIZE - 1)
TROWS = VOCAB + 8
CHUNKS = tuple([(c * 48, 48) for c in range(10)] + [(480, 32)])
GMAX = 48 + HALO


def _sc_copy(table8_hbm, out_hbm, buf0, buf1, gsem, ssem0, ssem1):
    wid = lax.axis_index("s") * N_CORES + lax.axis_index("c")
    rng = wid // 2
    half = wid % 2
    bufs = (buf0, buf1)
    ssems = (ssem0, ssem1)
    units = [(g, ci) for g in range(GSIZE) for ci in range(len(CHUNKS))]

    def gather(unit):
        g, ci = unit
        c0, _ch = CHUNKS[ci]
        m = GSIZE * half + g
        start = TROWS * m + (SEQ - HALO + c0) - 32 * rng - m
        return pltpu.async_copy(
            table8_hbm.at[pl.ds(pl.multiple_of(start, 8), _ch + HALO)],
            bufs[units.index(unit) % 2].at[pl.ds(0, _ch + HALO)],
            gsem,
        )

    pending = {}
    g = gather(units[0])
    for ui, unit in enumerate(units):
        gidx, ci = unit
        c0, ch = CHUNKS[ci]
        b = ui % 2
        g.wait()
        m = GSIZE * half + gidx
        handles = []
        for j in range(GSIZE):
            handles.append(pltpu.async_copy(
                bufs[b].at[pl.ds(HALO - 8 * j, ch)],
                out_hbm.at[32 * rng + m + 8 * j, pl.ds(c0, ch), :],
                ssems[b],
            ))
        pending[ui] = handles
        if ui + 1 < len(units):
            if ui - 1 >= 0:
                for h in pending.pop(ui - 1):
                    h.wait()
            g = gather(units[ui + 1])
    for hs in pending.values():
        for h in hs:
            h.wait()


def kernel(seq_len, emb):
    del seq_len
    idx = 1024 - (jnp.arange(8 * TROWS, dtype=jnp.int32) % TROWS)
    table8 = jnp.take(emb, jnp.maximum(idx, 0), axis=0)
    mesh = plsc.VectorSubcoreMesh(core_axis_name="c", subcore_axis_name="s")
    return pl.kernel(
        _sc_copy,
        mesh=mesh,
        out_type=jax.ShapeDtypeStruct((SEQ, SEQ, D_MODEL), jnp.float32),
        scratch_types=[
            pltpu.VMEM((GMAX, D_MODEL), jnp.float32),
            pltpu.VMEM((GMAX, D_MODEL), jnp.float32),
            pltpu.SemaphoreType.DMA,
            pltpu.SemaphoreType.DMA,
            pltpu.SemaphoreType.DMA,
        ],
    )(table8)


# speedup vs baseline: 1.2545x; 1.0185x over previous
import functools

import jax
import jax.numpy as jnp
from jax import lax
from jax.experimental import pallas as pl
from jax.experimental.pallas import tpu as pltpu
from jax.experimental.pallas import tpu_sc as plsc

D_MODEL = 768
SEQ = 512
VOCAB = 2 * SEQ + 1
N_CORES = 2
N_SUBCORES = 16
N_WORKERS = N_CORES * N_SUBCORES
I_PER_W = SEQ // N_WORKERS
N_PAIRS = I_PER_W // 2

GSIZE = 4
HALO = 8 * (GSIZE - 1)
TROWS = VOCAB + 8
CHUNKS = tuple([(0, 64)] + [(64 + c * 56, 56) for c in range(8)])
GA = 64 + HALO
GB = 56 + HALO


def _sc_copy(table8_hbm, out_hbm, buf0, buf1, gsem, ssem0, ssem1):
    wid = lax.axis_index("s") * N_CORES + lax.axis_index("c")
    rng = wid // 2
    half = wid % 2
    bufs = (buf0, buf1)
    ssems = (ssem0, ssem1)
    bigs = [(g, 0) for g in range(GSIZE)]
    smalls = [(g, ci) for g in range(GSIZE) for ci in range(1, len(CHUNKS))]
    units = []
    for k in range(GSIZE):
        units += [bigs[k], smalls[k]]
    units += smalls[GSIZE:]

    def gather(unit):
        g, ci = unit
        c0, _ch = CHUNKS[ci]
        m = GSIZE * half + g
        start = TROWS * m + (SEQ - HALO + c0) - 32 * rng - m
        return pltpu.async_copy(
            table8_hbm.at[pl.ds(pl.multiple_of(start, 8), _ch + HALO)],
            bufs[units.index(unit) % 2].at[pl.ds(0, _ch + HALO)],
            gsem,
        )

    assert all(CHUNKS[units[k][1]][1] <= 56 for k in range(1, len(units), 2))

    pending = {}
    g = gather(units[0])
    for ui, unit in enumerate(units):
        gidx, ci = unit
        c0, ch = CHUNKS[ci]
        b = ui % 2
        g.wait()
        m = GSIZE * half + gidx
        handles = []
        for j in range(GSIZE):
            handles.append(pltpu.async_copy(
                bufs[b].at[pl.ds(HALO - 8 * j, ch)],
                out_hbm.at[32 * rng + m + 8 * j, pl.ds(c0, ch), :],
                ssems[b],
            ))
        pending[ui] = handles
        if ui + 1 < len(units):
            if ui - 1 >= 0:
                for h in pending.pop(ui - 1):
                    h.wait()
            g = gather(units[ui + 1])
    for hs in pending.values():
        for h in hs:
            h.wait()


def kernel(seq_len, emb):
    del seq_len
    idx = 1024 - (jnp.arange(8 * TROWS, dtype=jnp.int32) % TROWS)
    table8 = jnp.take(emb, jnp.maximum(idx, 0), axis=0)
    mesh = plsc.VectorSubcoreMesh(core_axis_name="c", subcore_axis_name="s")
    return pl.kernel(
        _sc_copy,
        mesh=mesh,
        out_type=jax.ShapeDtypeStruct((SEQ, SEQ, D_MODEL), jnp.float32),
        scratch_types=[
            pltpu.VMEM((GA, D_MODEL), jnp.float32),
            pltpu.VMEM((GB, D_MODEL), jnp.float32),
            pltpu.SemaphoreType.DMA,
            pltpu.SemaphoreType.DMA,
            pltpu.SemaphoreType.DMA,
        ],
    )(table8)


# speedup vs baseline: 1.3002x; 1.0364x over previous
import functools

import jax
import jax.numpy as jnp
import numpy as np
from jax import lax
from jax.experimental import pallas as pl
from jax.experimental.pallas import tpu as pltpu
from jax.experimental.pallas import tpu_sc as plsc

D_MODEL = 768
SEQ = 512
VOCAB = 2 * SEQ + 1
N_CORES = 2
N_SUBCORES = 16
N_WORKERS = N_CORES * N_SUBCORES
I_PER_W = SEQ // N_WORKERS
N_PAIRS = I_PER_W // 2

GSIZE = 4
HALO = 8 * (GSIZE - 1)
TROWS = VOCAB + 8
CHUNKS = tuple([(0, 64)] + [(64 + c * 56, 56) for c in range(8)])
GA = 64 + HALO
GB = 56 + HALO


def _sc_copy(table8_hbm, out_hbm, buf0, buf1, gsem, ssem0, ssem1):
    wid = lax.axis_index("s") * N_CORES + lax.axis_index("c")
    rng = wid // 2
    half = wid % 2
    bufs = (buf0, buf1)
    ssems = (ssem0, ssem1)
    bigs = [(g, 0) for g in range(GSIZE)]
    smalls = [(g, ci) for g in range(GSIZE) for ci in range(1, len(CHUNKS))]
    units = []
    for k in range(GSIZE):
        units += [bigs[k], smalls[k]]
    units += smalls[GSIZE:]

    def gather(unit):
        g, ci = unit
        c0, _ch = CHUNKS[ci]
        m = GSIZE * half + g
        start = TROWS * m + (SEQ - HALO + c0) - 32 * rng - m
        return pltpu.async_copy(
            table8_hbm.at[pl.ds(pl.multiple_of(start, 8), _ch + HALO)],
            bufs[units.index(unit) % 2].at[pl.ds(0, _ch + HALO)],
            gsem,
        )

    assert all(CHUNKS[units[k][1]][1] <= 56 for k in range(1, len(units), 2))

    pending = {}
    g = gather(units[0])
    for ui, unit in enumerate(units):
        gidx, ci = unit
        c0, ch = CHUNKS[ci]
        b = ui % 2
        g.wait()
        m = GSIZE * half + gidx
        handles = []
        for j in range(GSIZE):
            handles.append(pltpu.async_copy(
                bufs[b].at[pl.ds(HALO - 8 * j, ch)],
                out_hbm.at[32 * rng + m + 8 * j, pl.ds(c0, ch), :],
                ssems[b],
            ))
        pending[ui] = handles
        if ui + 1 < len(units):
            if ui - 1 >= 0:
                for h in pending.pop(ui - 1):
                    h.wait()
            g = gather(units[ui + 1])
    for hs in pending.values():
        for h in hs:
            h.wait()


def kernel(seq_len, emb):
    del seq_len
    idx = np.maximum(1024 - (np.arange(8 * TROWS) % TROWS), 0).astype(np.int32)
    table8 = jnp.take(emb, jnp.asarray(idx), axis=0)
    mesh = plsc.VectorSubcoreMesh(core_axis_name="c", subcore_axis_name="s")
    return pl.kernel(
        _sc_copy,
        mesh=mesh,
        out_type=jax.ShapeDtypeStruct((SEQ, SEQ, D_MODEL), jnp.float32),
        scratch_types=[
            pltpu.VMEM((GA, D_MODEL), jnp.float32),
            pltpu.VMEM((GB, D_MODEL), jnp.float32),
            pltpu.SemaphoreType.DMA,
            pltpu.SemaphoreType.DMA,
            pltpu.SemaphoreType.DMA,
        ],
    )(table8)


# speedup vs baseline: 1.3028x; 1.0020x over previous
import jax
import jax.numpy as jnp
import numpy as np
from jax import lax
from jax.experimental import pallas as pl
from jax.experimental.pallas import tpu as pltpu
from jax.experimental.pallas import tpu_sc as plsc

D_MODEL = 768
SEQ = 512
VOCAB = 2 * SEQ + 1
N_CORES = 2
N_SUBCORES = 16
N_WORKERS = N_CORES * N_SUBCORES

GSIZE = 4
HALO = 8 * (GSIZE - 1)
TROWS = VOCAB + 8
CHUNKS = tuple([(0, 64)] + [(64 + c * 56, 56) for c in range(8)])
GA = 64 + HALO
GB = 56 + HALO


def _sc_copy(table8_hbm, out_hbm, buf0, buf1, gsem, ssem0, ssem1):
    wid = lax.axis_index("s") * N_CORES + lax.axis_index("c")
    rng = wid // 2
    half = wid % 2
    bufs = (buf0, buf1)
    ssems = (ssem0, ssem1)
    bigs = [(g, 0) for g in range(GSIZE)]
    smalls = [(g, ci) for g in range(GSIZE) for ci in range(1, len(CHUNKS))]
    units = []
    for k in range(GSIZE):
        units += [bigs[k], smalls[k]]
    units += smalls[GSIZE:]

    def gather(unit):
        g, ci = unit
        c0, _ch = CHUNKS[ci]
        m = GSIZE * half + g
        start = TROWS * m + (SEQ - HALO + c0) - 32 * rng - m
        return pltpu.async_copy(
            table8_hbm.at[pl.ds(pl.multiple_of(start, 8), _ch + HALO)],
            bufs[units.index(unit) % 2].at[pl.ds(0, _ch + HALO)],
            gsem,
        )

    assert all(CHUNKS[units[k][1]][1] <= 56 for k in range(1, len(units), 2))

    pending = {}
    g = gather(units[0])
    for ui, unit in enumerate(units):
        gidx, ci = unit
        c0, ch = CHUNKS[ci]
        b = ui % 2
        g.wait()
        m = GSIZE * half + gidx
        handles = []
        for j in range(GSIZE):
            handles.append(pltpu.async_copy(
                bufs[b].at[pl.ds(HALO - 8 * j, ch)],
                out_hbm.at[32 * rng + m + 8 * j, pl.ds(c0, ch), :],
                ssems[b],
            ))
        pending[ui] = handles
        if ui + 1 < len(units):
            if ui - 1 >= 0:
                for h in pending.pop(ui - 1):
                    h.wait()
            g = gather(units[ui + 1])
    for hs in pending.values():
        for h in hs:
            h.wait()


def kernel(seq_len, emb):
    del seq_len
    idx = np.maximum(1024 - (np.arange(8 * TROWS) % TROWS), 0).astype(np.int32)
    table8 = jnp.take(emb, jnp.asarray(idx), axis=0)
    mesh = plsc.VectorSubcoreMesh(core_axis_name="c", subcore_axis_name="s")
    return pl.kernel(
        _sc_copy,
        mesh=mesh,
        out_type=jax.ShapeDtypeStruct((SEQ, SEQ, D_MODEL), jnp.float32),
        scratch_types=[
            pltpu.VMEM((GA, D_MODEL), jnp.float32),
            pltpu.VMEM((GB, D_MODEL), jnp.float32),
            pltpu.SemaphoreType.DMA,
            pltpu.SemaphoreType.DMA,
            pltpu.SemaphoreType.DMA,
        ],
    )(table8)
